# Initial kernel scaffold; baseline (speedup 1.0000x reference)
import jax
import jax.numpy as jnp
from jax.experimental import pallas as pl


def kernel(x, indices, W1, b1, W2, b2):
    raise NotImplementedError("write your pallas kernel here")


# speedup vs baseline: 3.0927x; 3.0927x over previous
import functools

import jax
import jax.numpy as jnp
from jax import lax
from jax.experimental import pallas as pl
from jax.experimental.pallas import tpu as pltpu
from jax.experimental.pallas import tpu_sc as plsc

_B, _D, _H, _K = 4096, 512, 512, 16
_LANES = 16


def _build_mt_sparsecore(idx_flat, w1_flat):
    info = plsc.get_sparse_core_info()
    nw = info.num_cores * info.num_subcores
    th = _H // nw
    blk = _K * th
    rowlen = th * _D

    mesh = plsc.VectorSubcoreMesh(core_axis_name="c", subcore_axis_name="s")

    @functools.partial(
        pl.kernel,
        mesh=mesh,
        compiler_params=pltpu.CompilerParams(needs_layout_passes=False),
        out_type=jax.ShapeDtypeStruct((_H * _D,), jnp.float32),
        scratch_types=[
            pltpu.VMEM((blk,), jnp.int32),
            pltpu.VMEM((blk,), jnp.float32),
            pltpu.VMEM((rowlen,), jnp.float32),
        ],
    )
    def build(idx_hbm, w_hbm, m_hbm, idx_v, w_v, m_v):
        wid = lax.axis_index("s") * info.num_cores + lax.axis_index("c")
        pltpu.sync_copy(idx_hbm.at[pl.ds(wid * blk, blk)], idx_v)
        pltpu.sync_copy(w_hbm.at[pl.ds(wid * blk, blk)], w_v)

        def zero_chunk(i, c):
            m_v[pl.ds(i * _LANES, _LANES)] = jnp.zeros((_LANES,), jnp.float32)
            return c

        lax.fori_loop(0, rowlen // _LANES, zero_chunk, 0)

        row_off = lax.broadcasted_iota(jnp.int32, (_LANES,), 0) * _D
        for k in range(_K):
            addr = row_off + idx_v[pl.ds(k * _LANES, _LANES)]
            plsc.addupdate_scatter(m_v, [addr], w_v[pl.ds(k * _LANES, _LANES)])

        pltpu.sync_copy(m_v, m_hbm.at[pl.ds(wid * rowlen, rowlen)])

    return build(idx_flat, w1_flat)


def _forward_body(x_ref, mt_ref, b1_ref, w2t_ref, b2_ref, ha_ref, out_ref):
    h = lax.dot_general(
        x_ref[...],
        mt_ref[...],
        dimension_numbers=(((1,), (1,)), ((), ())),
        preferred_element_type=jnp.float32,
        precision=lax.Precision.HIGHEST,
    )
    ha = jnp.tanh(h + b1_ref[...])
    ha_ref[...] = ha
    o = jnp.dot(
        ha,
        w2t_ref[...],
        preferred_element_type=jnp.float32,
        precision=lax.Precision.HIGHEST,
    )
    out_ref[...] = jnp.tanh(o + b2_ref[...])


def _forward_tensorcore(x, mt, b1, w2, b2):
    bb = 512
    grid = (_B // bb,)
    ha, out = pl.pallas_call(
        _forward_body,
        grid=grid,
        in_specs=[
            pl.BlockSpec((bb, _D), lambda i: (i, 0)),
            pl.BlockSpec((_H, _D), lambda i: (0, 0)),
            pl.BlockSpec((1, _H), lambda i: (0, 0)),
            pl.BlockSpec((_H, 1), lambda i: (0, 0)),
            pl.BlockSpec((1, 1), lambda i: (0, 0)),
        ],
        out_specs=[
            pl.BlockSpec((bb, _H), lambda i: (i, 0)),
            pl.BlockSpec((bb, 1), lambda i: (i, 0)),
        ],
        out_shape=[
            jax.ShapeDtypeStruct((_B, _H), jnp.float32),
            jax.ShapeDtypeStruct((_B, 1), jnp.float32),
        ],
    )(x, mt, b1.reshape(1, _H), w2.reshape(_H, 1), b2.reshape(1, 1))
    return ha, out.reshape(_B)


def kernel(x, indices, W1, b1, W2, b2):
    info = plsc.get_sparse_core_info()
    nw = info.num_cores * info.num_subcores
    th = _H // nw
    idx_flat = (
        indices.T.astype(jnp.int32).reshape(_K, nw, th).transpose(1, 0, 2).reshape(-1)
    )
    w1_flat = (
        W1.T.astype(jnp.float32).reshape(_K, nw, th).transpose(1, 0, 2).reshape(-1)
    )
    mt = _build_mt_sparsecore(idx_flat, w1_flat).reshape(_H, _D)
    return _forward_tensorcore(x, mt, b1, W2, b2)


# speedup vs baseline: 4.5700x; 1.4777x over previous
import functools

import jax
import jax.numpy as jnp
from jax import lax
from jax.experimental import pallas as pl
from jax.experimental.pallas import tpu as pltpu
from jax.experimental.pallas import tpu_sc as plsc

_B, _D, _H, _K = 4096, 512, 512, 16
_LANES = 16


def _build_mt_sparsecore(idx_flat, w1_flat):
    info = plsc.get_sparse_core_info()
    nw = info.num_cores * info.num_subcores
    th = _H // nw
    blk = _K * th
    rowlen = th * _D

    mesh = plsc.VectorSubcoreMesh(core_axis_name="c", subcore_axis_name="s")

    @functools.partial(
        pl.kernel,
        mesh=mesh,
        compiler_params=pltpu.CompilerParams(needs_layout_passes=False),
        out_type=jax.ShapeDtypeStruct((_H * _D,), jnp.float32),
        scratch_types=[
            pltpu.VMEM((blk,), jnp.int32),
            pltpu.VMEM((blk,), jnp.float32),
            pltpu.VMEM((rowlen,), jnp.float32),
        ],
    )
    def build(idx_hbm, w_hbm, m_hbm, idx_v, w_v, m_v):
        wid = lax.axis_index("s") * info.num_cores + lax.axis_index("c")
        pltpu.sync_copy(idx_hbm.at[pl.ds(wid * blk, blk)], idx_v)
        pltpu.sync_copy(w_hbm.at[pl.ds(wid * blk, blk)], w_v)

        def zero_chunk(i, c):
            m_v[pl.ds(i * _LANES, _LANES)] = jnp.zeros((_LANES,), jnp.float32)
            return c

        lax.fori_loop(0, rowlen // _LANES, zero_chunk, 0)

        row_off = lax.broadcasted_iota(jnp.int32, (_LANES,), 0) * _D
        for k in range(_K):
            addr = row_off + idx_v[pl.ds(k * _LANES, _LANES)]
            plsc.addupdate_scatter(m_v, [addr], w_v[pl.ds(k * _LANES, _LANES)])

        pltpu.sync_copy(m_v, m_hbm.at[pl.ds(wid * rowlen, rowlen)])

    return build(idx_flat, w1_flat)


def _forward_body(x_ref, mt_ref, b1_ref, w2_ref, b2_ref, ha_ref, out_ref):
    h = lax.dot_general(
        x_ref[...],
        mt_ref[...],
        dimension_numbers=(((1,), (1,)), ((), ())),
        preferred_element_type=jnp.float32,
    )
    ha = jnp.tanh(h + b1_ref[...])
    ha_ref[...] = ha
    o = jnp.sum(ha * w2_ref[...], axis=1, keepdims=True)
    out_ref[...] = jnp.tanh(o + b2_ref[...])


def _forward_tensorcore(x, mt, b1, w2, b2):
    bb = 512
    grid = (_B // bb,)
    ha, out = pl.pallas_call(
        _forward_body,
        grid=grid,
        in_specs=[
            pl.BlockSpec((bb, _D), lambda i: (i, 0)),
            pl.BlockSpec((_H, _D), lambda i: (0, 0)),
            pl.BlockSpec((1, _H), lambda i: (0, 0)),
            pl.BlockSpec((1, _H), lambda i: (0, 0)),
            pl.BlockSpec((1, 1), lambda i: (0, 0)),
        ],
        out_specs=[
            pl.BlockSpec((bb, _H), lambda i: (i, 0)),
            pl.BlockSpec((bb, 1), lambda i: (i, 0)),
        ],
        out_shape=[
            jax.ShapeDtypeStruct((_B, _H), jnp.float32),
            jax.ShapeDtypeStruct((_B, 1), jnp.float32),
        ],
    )(x, mt, b1.reshape(1, _H), w2.reshape(1, _H), b2.reshape(1, 1))
    return ha, out.reshape(_B)


def kernel(x, indices, W1, b1, W2, b2):
    info = plsc.get_sparse_core_info()
    nw = info.num_cores * info.num_subcores
    th = _H // nw
    idx_flat = (
        indices.T.astype(jnp.int32).reshape(_K, nw, th).transpose(1, 0, 2).reshape(-1)
    )
    w1_flat = (
        W1.T.astype(jnp.float32).reshape(_K, nw, th).transpose(1, 0, 2).reshape(-1)
    )
    mt = _build_mt_sparsecore(idx_flat, w1_flat).reshape(_H, _D)
    return _forward_tensorcore(x, mt, b1, W2, b2)
